# baseline (speedup 1.0000x reference)
import jax
import jax.numpy as jnp
from jax import lax
from jax.experimental import pallas as pl
from jax.experimental.pallas import tpu as pltpu
from jax.experimental.pallas import tpu_sc as plsc

N_NODES = 10000
IN_CH = 128
OUT_CH = 64
HALF = OUT_CH // 2

NC = 2
NS = 16
CHUNK = 128
N_GARBAGE = 240
N_ROWS = N_NODES + N_GARBAGE
RPT = N_ROWS // NS
DEG_W = 8
LANES = 16
NBUF = 10
LEAD = 8
EROWS = 160


def _sc_mesh():
    return plsc.VectorSubcoreMesh(core_axis_name="c", subcore_axis_name="s")


def _sc_params():
    return pltpu.CompilerParams(
        use_tc_tiling_on_sc=False,
        disable_bounds_checks=True,
        disable_semaphore_checks=True,
    )


def _deg_body(dst_hbm, ones_hbm, zeros_hbm, out_hbm, dst_v, ones_v, acc,
              ssem):
    c = lax.axis_index("c")
    s = lax.axis_index("s")
    nchunk = dst_hbm.shape[1] // NC
    pltpu.sync_copy(dst_hbm.at[s, pl.ds(c * nchunk, nchunk)], dst_v)
    pltpu.sync_copy(ones_hbm, ones_v)
    pltpu.sync_copy(zeros_hbm, acc.at[pl.ds(s * RPT, RPT)])
    plsc.subcore_barrier()

    def body(i, _):
        for b in range(NBUF):
            j = i * NBUF + b

            @pl.when(j >= NBUF)
            def _drain():
                pltpu.make_async_copy(ones_v, acc.at[dst_v.at[j - NBUF]],
                                      ssem.at[b]).wait()
            pltpu.async_copy(ones_v, acc.at[dst_v.at[j]], ssem.at[b],
                             add=True)
        return _

    lax.fori_loop(0, nchunk // NBUF, body, None)
    for b in range(NBUF):
        pltpu.make_async_copy(ones_v, acc.at[dst_v.at[nchunk - NBUF + b]],
                              ssem.at[b]).wait()
    plsc.subcore_barrier()
    pltpu.sync_copy(acc.at[pl.ds(s * RPT, RPT)],
                    out_hbm.at[c, pl.ds(s * RPT, RPT)])


def _hop_body(table_hbm, src_hbm, dst_hbm, scale_hbm, zeros_hbm, out_hbm,
              src_v, dst_v, buf, abuf, tbuf, sbuf, acc, gsem, ssem):
    c = lax.axis_index("c")
    s = lax.axis_index("s")
    nchunk = src_hbm.shape[2]
    pltpu.sync_copy(src_hbm.at[c, s], src_v)
    pltpu.sync_copy(dst_hbm.at[s], dst_v)
    pltpu.sync_copy(zeros_hbm, acc.at[pl.ds(s * RPT, RPT)])
    plsc.subcore_barrier()

    def gather(j, b):
        pltpu.async_copy(table_hbm.at[src_v.at[j]], buf.at[b], gsem.at[b])

    def gather_wait(j, b):
        pltpu.make_async_copy(table_hbm.at[src_v.at[j]], buf.at[b],
                              gsem.at[b]).wait()

    def scatter(j, b):
        pltpu.async_copy(buf.at[b], acc.at[dst_v.at[j]], ssem.at[b], add=True)

    def scatter_wait(j, b):
        pltpu.make_async_copy(buf.at[b], acc.at[dst_v.at[j]],
                              ssem.at[b]).wait()

    for jj in range(LEAD):
        gather(jj, jj)

    def body(i, _):
        for b in range(NBUF):
            j = i * NBUF + b
            b2 = (b + LEAD) % NBUF
            gather_wait(j, b)
            scatter(j, b)

            @pl.when(j + LEAD < nchunk)
            def _issue():
                @pl.when(j + LEAD >= NBUF)
                def _drain():
                    scatter_wait(j + LEAD - NBUF, b2)
                gather(j + LEAD, b2)
        return _

    lax.fori_loop(0, nchunk // NBUF, body, None)
    for b in range(NBUF):
        scatter_wait(nchunk - NBUF + b, b)
    plsc.subcore_barrier()

    def epi(p, _):
        r0 = s * RPT + p * EROWS
        pltpu.sync_copy(acc.at[pl.ds(r0, EROWS)], abuf)
        pltpu.sync_copy(table_hbm.at[pl.ds(c * N_ROWS + r0, EROWS)], tbuf)
        pltpu.sync_copy(scale_hbm.at[pl.ds(r0, EROWS)], sbuf)

        def rows(i, _):
            for k in range(HALF // LANES):
                sl = (i, pl.ds(k * LANES, LANES))
                abuf[sl] = (abuf[sl] + tbuf[sl]) * sbuf[sl]
            return _

        lax.fori_loop(0, EROWS, rows, None)
        pltpu.sync_copy(abuf, out_hbm.at[c, pl.ds(r0, EROWS)])
        return _

    lax.fori_loop(0, RPT // EROWS, epi, None)


def _deg_call(dst_tiles):
    kfn = pl.kernel(
        _deg_body,
        out_type=jax.ShapeDtypeStruct((NC, N_ROWS, DEG_W), jnp.float32),
        mesh=_sc_mesh(),
        compiler_params=_sc_params(),
        scratch_types=[
            pltpu.VMEM((dst_tiles.shape[1] // NC, CHUNK), jnp.int32),
            pltpu.VMEM((CHUNK, DEG_W), jnp.float32),
            pltpu.VMEM_SHARED((N_ROWS, DEG_W), jnp.float32),
            pltpu.SemaphoreType.DMA((NBUF,)),
        ],
    )
    return kfn(dst_tiles, jnp.ones((CHUNK, DEG_W), jnp.float32),
               jnp.zeros((RPT, DEG_W), jnp.float32))


def _hop_call(table, src_tiles, dst_tiles, scale):
    kfn = pl.kernel(
        _hop_body,
        out_type=jax.ShapeDtypeStruct((NC, N_ROWS, HALF), jnp.float32),
        mesh=_sc_mesh(),
        compiler_params=_sc_params(),
        scratch_types=[
            pltpu.VMEM(src_tiles.shape[2:], jnp.int32),
            pltpu.VMEM(dst_tiles.shape[1:], jnp.int32),
            pltpu.VMEM((NBUF, CHUNK, HALF), jnp.float32),
            pltpu.VMEM((EROWS, HALF), jnp.float32),
            pltpu.VMEM((EROWS, HALF), jnp.float32),
            pltpu.VMEM((EROWS, HALF), jnp.float32),
            pltpu.VMEM_SHARED((N_ROWS, HALF), jnp.float32),
            pltpu.SemaphoreType.DMA((NBUF,)),
            pltpu.SemaphoreType.DMA((NBUF,)),
        ],
    )
    out = kfn(table, src_tiles, dst_tiles, scale,
              jnp.zeros((RPT, HALF), jnp.float32))
    return out.reshape(NC * N_ROWS, HALF)


def _prep_tc(x_ref, w_ref, degp_ref, v0_ref, dinvx_ref, disx_ref):
    deg = degp_ref[0, :, 0:1] + degp_ref[1, :, 0:1] + 1.0
    valid = lax.broadcasted_iota(jnp.int32, (N_ROWS, 1), 0) < N_NODES
    dis = jnp.where(valid, lax.rsqrt(deg), 0.0)
    dinv = jnp.where(valid, 1.0 / deg, 0.0)
    dinvx_ref[...] = jnp.broadcast_to(dinv, (N_ROWS, HALF))
    disx_ref[...] = jnp.broadcast_to(dis, (N_ROWS, HALF))
    g = lax.dot_general(x_ref[...], w_ref[...],
                        (((1,), (1,)), ((), ())),
                        preferred_element_type=jnp.float32)
    gs = g * dis[:N_NODES]
    zpad = jnp.zeros((N_GARBAGE, HALF), jnp.float32)
    v0_ref[pl.ds(0, N_NODES), :] = gs[:, :HALF]
    v0_ref[pl.ds(N_NODES, N_GARBAGE), :] = zpad
    v0_ref[pl.ds(N_ROWS, N_NODES), :] = gs[:, HALF:]
    v0_ref[pl.ds(N_ROWS + N_NODES, N_GARBAGE), :] = zpad


def _final_tc(p_ref, b_ref, out_ref):
    h = jnp.concatenate(
        [p_ref[:N_NODES, :], p_ref[N_ROWS:N_ROWS + N_NODES, :]], axis=1)
    logits = h + b_ref[...]
    m = jnp.max(logits, axis=-1, keepdims=True)
    lse = jnp.log(jnp.sum(jnp.exp(logits - m), axis=-1, keepdims=True)) + m
    out_ref[...] = logits - lse


def kernel(x, edge_index, W, b):
    n_edges = edge_index.shape[1]
    ept = n_edges // NS
    nchunk = -(-ept // CHUNK)
    nchunk += (-nchunk) % (2 * NBUF)
    pad_per_tile = nchunk * CHUNK - ept

    src = edge_index[0]
    dst = edge_index[1]
    ar = jnp.arange(NS * pad_per_tile, dtype=jnp.int32)
    pad_src = (ar * 131) % N_NODES
    pad_dst = N_NODES + (ar % N_GARBAGE)
    src_t = jnp.concatenate(
        [src.reshape(NS, ept), pad_src.reshape(NS, pad_per_tile)], axis=1
    ).reshape(NS, nchunk, CHUNK)
    src_tiles = jnp.stack([src_t, src_t + N_ROWS])
    dst_tiles = jnp.concatenate(
        [dst.reshape(NS, ept), pad_dst.reshape(NS, pad_per_tile)], axis=1
    ).reshape(NS, nchunk, CHUNK)

    degp = _deg_call(dst_tiles)

    f32 = jnp.float32
    v0, dinvx, disx = pl.pallas_call(
        _prep_tc,
        out_shape=[
            jax.ShapeDtypeStruct((NC * N_ROWS, HALF), f32),
            jax.ShapeDtypeStruct((N_ROWS, HALF), f32),
            jax.ShapeDtypeStruct((N_ROWS, HALF), f32),
        ],
    )(x, W, degp)

    v2 = _hop_call(v0, src_tiles, dst_tiles, dinvx)
    v4 = _hop_call(v2, src_tiles, dst_tiles, disx)

    out = pl.pallas_call(
        _final_tc,
        out_shape=jax.ShapeDtypeStruct((N_NODES, OUT_CH), f32),
    )(v4, b.reshape(1, OUT_CH))

    return out


# speedup vs baseline: 1.0025x; 1.0025x over previous
import jax
import jax.numpy as jnp
from jax import lax
from jax.experimental import pallas as pl
from jax.experimental.pallas import tpu as pltpu
from jax.experimental.pallas import tpu_sc as plsc

N_NODES = 10000
IN_CH = 128
OUT_CH = 64
HALF = OUT_CH // 2

NC = 2
NS = 16
CHUNK = 128
N_GARBAGE = 240
N_ROWS = N_NODES + N_GARBAGE
RPT = N_ROWS // NS
DEG_W = 8
LANES = 16
NBUF = 8
LEAD = 6
EROWS = 160


def _sc_mesh():
    return plsc.VectorSubcoreMesh(core_axis_name="c", subcore_axis_name="s")


def _sc_params():
    return pltpu.CompilerParams(
        use_tc_tiling_on_sc=False,
        disable_bounds_checks=True,
        disable_semaphore_checks=True,
    )


def _deg_body(dst_hbm, ones_hbm, zeros_hbm, out_hbm, dst_v, ones_v, acc,
              ssem):
    c = lax.axis_index("c")
    s = lax.axis_index("s")
    nchunk = dst_hbm.shape[1] // NC
    pltpu.sync_copy(dst_hbm.at[s, pl.ds(c * nchunk, nchunk)], dst_v)
    pltpu.sync_copy(ones_hbm, ones_v)
    pltpu.sync_copy(zeros_hbm, acc.at[pl.ds(s * RPT, RPT)])
    plsc.subcore_barrier()

    def body(i, _):
        for b in range(NBUF):
            j = i * NBUF + b

            @pl.when(j >= NBUF)
            def _drain():
                pltpu.make_async_copy(ones_v, acc.at[dst_v.at[j - NBUF]],
                                      ssem.at[b]).wait()
            pltpu.async_copy(ones_v, acc.at[dst_v.at[j]], ssem.at[b],
                             add=True)
        return _

    lax.fori_loop(0, nchunk // NBUF, body, None)
    for b in range(NBUF):
        pltpu.make_async_copy(ones_v, acc.at[dst_v.at[nchunk - NBUF + b]],
                              ssem.at[b]).wait()
    plsc.subcore_barrier()
    pltpu.sync_copy(acc.at[pl.ds(s * RPT, RPT)],
                    out_hbm.at[c, pl.ds(s * RPT, RPT)])


def _hop_body(table_hbm, src_hbm, dst_hbm, scale_hbm, zeros_hbm, out_hbm,
              src_v, dst_v, buf, abuf, tbuf, sbuf, acc, gsem, ssem):
    c = lax.axis_index("c")
    s = lax.axis_index("s")
    nchunk = src_hbm.shape[2]
    pltpu.sync_copy(src_hbm.at[c, s], src_v)
    pltpu.sync_copy(dst_hbm.at[s], dst_v)
    pltpu.sync_copy(zeros_hbm, acc.at[pl.ds(s * RPT, RPT)])
    plsc.subcore_barrier()

    def gather(j, b):
        pltpu.async_copy(table_hbm.at[src_v.at[j]], buf.at[b], gsem.at[b])

    def gather_wait(j, b):
        pltpu.make_async_copy(table_hbm.at[src_v.at[j]], buf.at[b],
                              gsem.at[b]).wait()

    def scatter(j, b):
        pltpu.async_copy(buf.at[b], acc.at[dst_v.at[j]], ssem.at[b], add=True)

    def scatter_wait(j, b):
        pltpu.make_async_copy(buf.at[b], acc.at[dst_v.at[j]],
                              ssem.at[b]).wait()

    for jj in range(LEAD):
        gather(jj, jj)

    def body(i, _):
        for b in range(NBUF):
            j = i * NBUF + b
            b2 = (b + LEAD) % NBUF
            gather_wait(j, b)
            scatter(j, b)

            @pl.when(j + LEAD < nchunk)
            def _issue():
                @pl.when(j + LEAD >= NBUF)
                def _drain():
                    scatter_wait(j + LEAD - NBUF, b2)
                gather(j + LEAD, b2)
        return _

    lax.fori_loop(0, nchunk // NBUF, body, None)
    for b in range(NBUF):
        scatter_wait(nchunk - NBUF + b, b)
    plsc.subcore_barrier()

    def epi(p, _):
        r0 = s * RPT + p * EROWS
        pltpu.sync_copy(acc.at[pl.ds(r0, EROWS)], abuf)
        pltpu.sync_copy(table_hbm.at[pl.ds(c * N_ROWS + r0, EROWS)], tbuf)
        pltpu.sync_copy(scale_hbm.at[pl.ds(r0, EROWS)], sbuf)

        def rows(i, _):
            for k in range(HALF // LANES):
                sl = (i, pl.ds(k * LANES, LANES))
                abuf[sl] = (abuf[sl] + tbuf[sl]) * sbuf[sl]
            return _

        lax.fori_loop(0, EROWS, rows, None)
        pltpu.sync_copy(abuf, out_hbm.at[c, pl.ds(r0, EROWS)])
        return _

    lax.fori_loop(0, RPT // EROWS, epi, None)


def _deg_call(dst_tiles):
    kfn = pl.kernel(
        _deg_body,
        out_type=jax.ShapeDtypeStruct((NC, N_ROWS, DEG_W), jnp.float32),
        mesh=_sc_mesh(),
        compiler_params=_sc_params(),
        scratch_types=[
            pltpu.VMEM((dst_tiles.shape[1] // NC, CHUNK), jnp.int32),
            pltpu.VMEM((CHUNK, DEG_W), jnp.float32),
            pltpu.VMEM_SHARED((N_ROWS, DEG_W), jnp.float32),
            pltpu.SemaphoreType.DMA((NBUF,)),
        ],
    )
    return kfn(dst_tiles, jnp.ones((CHUNK, DEG_W), jnp.float32),
               jnp.zeros((RPT, DEG_W), jnp.float32))


def _hop_call(table, src_tiles, dst_tiles, scale):
    kfn = pl.kernel(
        _hop_body,
        out_type=jax.ShapeDtypeStruct((NC, N_ROWS, HALF), jnp.float32),
        mesh=_sc_mesh(),
        compiler_params=_sc_params(),
        scratch_types=[
            pltpu.VMEM(src_tiles.shape[2:], jnp.int32),
            pltpu.VMEM(dst_tiles.shape[1:], jnp.int32),
            pltpu.VMEM((NBUF, CHUNK, HALF), jnp.float32),
            pltpu.VMEM((EROWS, HALF), jnp.float32),
            pltpu.VMEM((EROWS, HALF), jnp.float32),
            pltpu.VMEM((EROWS, HALF), jnp.float32),
            pltpu.VMEM_SHARED((N_ROWS, HALF), jnp.float32),
            pltpu.SemaphoreType.DMA((NBUF,)),
            pltpu.SemaphoreType.DMA((NBUF,)),
        ],
    )
    out = kfn(table, src_tiles, dst_tiles, scale,
              jnp.zeros((RPT, HALF), jnp.float32))
    return out.reshape(NC * N_ROWS, HALF)


def _prep_tc(x_ref, w_ref, degp_ref, v0_ref, dinvx_ref, disx_ref):
    deg = degp_ref[0, :, 0:1] + degp_ref[1, :, 0:1] + 1.0
    valid = lax.broadcasted_iota(jnp.int32, (N_ROWS, 1), 0) < N_NODES
    dis = jnp.where(valid, lax.rsqrt(deg), 0.0)
    dinv = jnp.where(valid, 1.0 / deg, 0.0)
    dinvx_ref[...] = jnp.broadcast_to(dinv, (N_ROWS, HALF))
    disx_ref[...] = jnp.broadcast_to(dis, (N_ROWS, HALF))
    g = lax.dot_general(x_ref[...], w_ref[...],
                        (((1,), (1,)), ((), ())),
                        preferred_element_type=jnp.float32)
    gs = g * dis[:N_NODES]
    zpad = jnp.zeros((N_GARBAGE, HALF), jnp.float32)
    v0_ref[pl.ds(0, N_NODES), :] = gs[:, :HALF]
    v0_ref[pl.ds(N_NODES, N_GARBAGE), :] = zpad
    v0_ref[pl.ds(N_ROWS, N_NODES), :] = gs[:, HALF:]
    v0_ref[pl.ds(N_ROWS + N_NODES, N_GARBAGE), :] = zpad


def _final_tc(p_ref, b_ref, out_ref):
    h = jnp.concatenate(
        [p_ref[:N_NODES, :], p_ref[N_ROWS:N_ROWS + N_NODES, :]], axis=1)
    logits = h + b_ref[...]
    m = jnp.max(logits, axis=-1, keepdims=True)
    lse = jnp.log(jnp.sum(jnp.exp(logits - m), axis=-1, keepdims=True)) + m
    out_ref[...] = logits - lse


def kernel(x, edge_index, W, b):
    n_edges = edge_index.shape[1]
    ept = n_edges // NS
    nchunk = -(-ept // CHUNK)
    nchunk += (-nchunk) % (2 * NBUF)
    pad_per_tile = nchunk * CHUNK - ept

    src = edge_index[0]
    dst = edge_index[1]
    ar = jnp.arange(NS * pad_per_tile, dtype=jnp.int32)
    pad_src = (ar * 131) % N_NODES
    pad_dst = N_NODES + (ar % N_GARBAGE)
    src_t = jnp.concatenate(
        [src.reshape(NS, ept), pad_src.reshape(NS, pad_per_tile)], axis=1
    ).reshape(NS, nchunk, CHUNK)
    src_tiles = jnp.stack([src_t, src_t + N_ROWS])
    dst_tiles = jnp.concatenate(
        [dst.reshape(NS, ept), pad_dst.reshape(NS, pad_per_tile)], axis=1
    ).reshape(NS, nchunk, CHUNK)

    degp = _deg_call(dst_tiles)

    f32 = jnp.float32
    v0, dinvx, disx = pl.pallas_call(
        _prep_tc,
        out_shape=[
            jax.ShapeDtypeStruct((NC * N_ROWS, HALF), f32),
            jax.ShapeDtypeStruct((N_ROWS, HALF), f32),
            jax.ShapeDtypeStruct((N_ROWS, HALF), f32),
        ],
    )(x, W, degp)

    v2 = _hop_call(v0, src_tiles, dst_tiles, dinvx)
    v4 = _hop_call(v2, src_tiles, dst_tiles, disx)

    out = pl.pallas_call(
        _final_tc,
        out_shape=jax.ShapeDtypeStruct((N_NODES, OUT_CH), f32),
    )(v4, b.reshape(1, OUT_CH))

    return out
